# baseline (speedup 1.0000x reference)
import jax
import jax.numpy as jnp
from jax import lax
from jax.experimental import pallas as pl
from jax.experimental.pallas import tpu as pltpu, tpu_sc as plsc

NC = 2
NS = 16
L = 16
NW = NC * NS
B = 16384
F = 16
BPW = B // NW
CHUNK = 128
NCHUNK = BPW // CHUNK
G = 8
NG = BPW // G
NPAR = 4
RT = 7813
RTP = RT * 128
HALF = RT * 1024
NFLAT = 2 * HALF

NT_W = 245
NCH = 5
CT = 49


def _linearize_body(p3_hbm, q3_hbm, pout_hbm, qout_hbm, sem):
    wid = lax.axis_index("s") * NC + lax.axis_index("c")
    t0 = wid * NT_W
    descs = []
    for (src3, dst3) in ((p3_hbm, pout_hbm), (q3_hbm, qout_hbm)):
        for g in range(2):
            for c in range(NCH):
                start = jnp.minimum(t0 + c * CT, RT - CT)
                s128 = pl.multiple_of(start * 128, 128)
                descs.append(pltpu.async_copy(
                    src3.at[g, :, pl.ds(s128, CT * 128)],
                    dst3.at[g, :, pl.ds(s128, CT * 128)],
                    sem))
    for d in descs:
        d.wait()


def _gather_body(u_hbm, i_hbm, pflat_hbm, qflat_hbm, bu_hbm, bi_hbm, out_hbm,
                 uidx_v, iidx_v, pidx_v, qidx_v, pbuf_v, qbuf_v,
                 bu_v, bi_v, out_v, sem0, sem1, sem2, sem3, gsem):
    wid = lax.axis_index("s") * NC + lax.axis_index("c")
    base = wid * BPW
    pltpu.sync_copy(u_hbm.at[pl.ds(base, BPW)], uidx_v.at[pl.ds(0, BPW)])
    pltpu.sync_copy(i_hbm.at[pl.ds(base, BPW)], iidx_v.at[pl.ds(0, BPW)])

    gdescs = []
    for j in range(NCHUNK):
        s = pl.ds(j * CHUNK, CHUNK)
        gdescs.append(pltpu.async_copy(bu_hbm.at[uidx_v.at[s]], bu_v.at[s], gsem))
        gdescs.append(pltpu.async_copy(bi_hbm.at[iidx_v.at[s]], bi_v.at[s], gsem))

    lane = lax.iota(jnp.int32, L)
    K = (lane >> 3) * HALF + (lane & 7) * 128
    sems = (sem0, sem1, sem2, sem3)

    def fire(g, par):
        uv = uidx_v[pl.ds(g * G, L)]
        iv = iidx_v[pl.ds(g * G, L)]
        pr = pidx_v.at[par]
        qr = qidx_v.at[par]
        for j in range(G):
            pr[pl.ds(j * L, L)] = K + ((uv[j] >> 7) * 1024 + (uv[j] & 127))
            qr[pl.ds(j * L, L)] = K + ((iv[j] >> 7) * 1024 + (iv[j] & 127))
        sem = sems[par]
        pltpu.async_copy(pflat_hbm.at[pidx_v.at[par]], pbuf_v.at[par], sem)
        pltpu.async_copy(qflat_hbm.at[qidx_v.at[par]], qbuf_v.at[par], sem)

    def proc(g, par, half, acc):
        sem = sems[par]
        pltpu.make_async_copy(pflat_hbm.at[pl.ds(0, G * L)], pbuf_v.at[par],
                              sem).wait()
        pltpu.make_async_copy(qflat_hbm.at[pl.ds(0, G * L)], qbuf_v.at[par],
                              sem).wait()
        pb = pbuf_v.at[par]
        qb = qbuf_v.at[par]
        for j in range(G):
            pv = pb[pl.ds(j * L, L)]
            qv = qb[pl.ds(j * L, L)]
            s = jnp.sum(pv * qv)
            acc = jnp.where(lane == half + j, acc + s, acc)
        return acc

    for d in gdescs:
        d.wait()

    for p in range(NPAR - 1):
        fire(p, p)

    def quad(k, carry):
        g0 = 4 * k
        blk0 = pl.ds(2 * k * L, L)
        blk1 = pl.ds((2 * k + 1) * L, L)
        acc0 = bu_v[blk0] + bi_v[blk0]
        acc1 = bu_v[blk1] + bi_v[blk1]
        for jj in range(4):
            g = g0 + jj

            @pl.when(g + NPAR - 1 < NG)
            def _(g=g, jj=jj):
                fire(g + NPAR - 1, (jj + NPAR - 1) % NPAR)

            if jj < 2:
                acc0 = proc(g, jj, (jj % 2) * G, acc0)
            else:
                acc1 = proc(g, jj, (jj % 2) * G, acc1)
        out_v[blk0] = acc0
        out_v[blk1] = acc1
        return carry

    lax.fori_loop(0, NG // 4, quad, 0)
    pltpu.sync_copy(out_v, out_hbm.at[pl.ds(base, BPW)])


def kernel(user_id, item_id, P, Q, Bu, Bi):
    u = user_id.reshape(-1)
    i = item_id.reshape(-1)
    bu = Bu.reshape(-1)
    bi = Bi.reshape(-1)
    p3 = P.T.reshape(2, 8, 1000000)
    q3 = Q.T.reshape(2, 8, 1000000)
    mesh = plsc.VectorSubcoreMesh(core_axis_name="c", subcore_axis_name="s",
                                  num_cores=NC, num_subcores=NS)
    cp = pltpu.CompilerParams(needs_layout_passes=False)

    pout, qout = pl.kernel(
        _linearize_body,
        out_type=(jax.ShapeDtypeStruct((2, 8, RTP), jnp.float32),
                  jax.ShapeDtypeStruct((2, 8, RTP), jnp.float32)),
        mesh=mesh,
        compiler_params=cp,
        scratch_types=[pltpu.SemaphoreType.DMA],
    )(p3, q3)

    def flat(t):
        return (t.reshape(2, 8, RT, 128)
                .transpose(0, 2, 1, 3)
                .reshape(NFLAT))

    out = pl.kernel(
        _gather_body,
        out_type=jax.ShapeDtypeStruct((B,), jnp.float32),
        mesh=mesh,
        compiler_params=cp,
        scratch_types=[
            pltpu.VMEM((BPW + L,), jnp.int32),
            pltpu.VMEM((BPW + L,), jnp.int32),
            pltpu.VMEM((NPAR, G * L), jnp.int32),
            pltpu.VMEM((NPAR, G * L), jnp.int32),
            pltpu.VMEM((NPAR, G * L), jnp.float32),
            pltpu.VMEM((NPAR, G * L), jnp.float32),
            pltpu.VMEM((BPW,), jnp.float32),
            pltpu.VMEM((BPW,), jnp.float32),
            pltpu.VMEM((BPW,), jnp.float32),
            pltpu.SemaphoreType.DMA,
            pltpu.SemaphoreType.DMA,
            pltpu.SemaphoreType.DMA,
            pltpu.SemaphoreType.DMA,
            pltpu.SemaphoreType.DMA,
        ],
    )(u, i, flat(pout), flat(qout), bu, bi)
    return out.reshape(B, 1)


# speedup vs baseline: 28.1890x; 28.1890x over previous
import jax
import jax.numpy as jnp
from jax import lax
from jax.experimental import pallas as pl
from jax.experimental.pallas import tpu as pltpu, tpu_sc as plsc

NC = 2
NS = 16
L = 16
NW = NC * NS
B = 16384
F = 16
BPW = B // NW
CHUNK = 128
NCHUNK = BPW // CHUNK
G = 8
NG = BPW // G
NPAR = 4
RT = 7813
RTP = RT * 128
HALF = RT * 1024
NFLAT = 2 * HALF

NT_W = 245
NCH = 5
CT = 49


def _linearize_body(p3_hbm, q3_hbm, pout_hbm, qout_hbm, buf_v,
                    isem0, isem1, osem0, osem1):
    wid = lax.axis_index("s") * NC + lax.axis_index("c")
    t0 = wid * NT_W
    isems = (isem0, isem1)
    osems = (osem0, osem1)
    chunks = []
    for (src3, dst3) in ((p3_hbm, pout_hbm), (q3_hbm, qout_hbm)):
        for g in range(2):
            for c in range(NCH):
                chunks.append((src3, dst3, g, c))

    def s128_of(c):
        start = jnp.minimum(t0 + c * CT, RT - CT)
        return pl.multiple_of(start * 128, 128)

    def start_in(idx, par):
        src3, _, g, c = chunks[idx]
        pltpu.async_copy(src3.at[g, :, pl.ds(s128_of(c), CT * 128)],
                         buf_v.at[par], isems[par])

    def wait_in(par):
        pltpu.make_async_copy(p3_hbm.at[0, :, pl.ds(0, CT * 128)],
                              buf_v.at[par], isems[par]).wait()

    def start_out(idx, par):
        _, dst3, g, c = chunks[idx]
        pltpu.async_copy(buf_v.at[par],
                         dst3.at[g, :, pl.ds(s128_of(c), CT * 128)],
                         osems[par])

    def wait_out(par):
        pltpu.make_async_copy(buf_v.at[par],
                              pout_hbm.at[0, :, pl.ds(0, CT * 128)],
                              osems[par]).wait()

    n = len(chunks)
    start_in(0, 0)
    for c in range(n):
        par = c & 1
        wait_in(par)
        start_out(c, par)
        if c + 1 < n:
            if c >= 1:
                wait_out(1 - par)
            start_in(c + 1, 1 - par)
    wait_out(0 if (n - 1) & 1 == 0 else 1)
    wait_out(1 if (n - 1) & 1 == 0 else 0)


def _gather_body(u_hbm, i_hbm, pflat_hbm, qflat_hbm, bu_hbm, bi_hbm, out_hbm,
                 uidx_v, iidx_v, pidx_v, qidx_v, pbuf_v, qbuf_v,
                 bu_v, bi_v, out_v, sem0, sem1, sem2, sem3, gsem):
    wid = lax.axis_index("s") * NC + lax.axis_index("c")
    base = wid * BPW
    pltpu.sync_copy(u_hbm.at[pl.ds(base, BPW)], uidx_v.at[pl.ds(0, BPW)])
    pltpu.sync_copy(i_hbm.at[pl.ds(base, BPW)], iidx_v.at[pl.ds(0, BPW)])

    gdescs = []
    for j in range(NCHUNK):
        s = pl.ds(j * CHUNK, CHUNK)
        gdescs.append(pltpu.async_copy(bu_hbm.at[uidx_v.at[s]], bu_v.at[s], gsem))
        gdescs.append(pltpu.async_copy(bi_hbm.at[iidx_v.at[s]], bi_v.at[s], gsem))

    lane = lax.iota(jnp.int32, L)
    K = (lane >> 3) * HALF + (lane & 7) * 128
    sems = (sem0, sem1, sem2, sem3)

    def fire(g, par):
        uv = uidx_v[pl.ds(g * G, L)]
        iv = iidx_v[pl.ds(g * G, L)]
        pr = pidx_v.at[par]
        qr = qidx_v.at[par]
        for j in range(G):
            pr[pl.ds(j * L, L)] = K + ((uv[j] >> 7) * 1024 + (uv[j] & 127))
            qr[pl.ds(j * L, L)] = K + ((iv[j] >> 7) * 1024 + (iv[j] & 127))
        sem = sems[par]
        pltpu.async_copy(pflat_hbm.at[pidx_v.at[par]], pbuf_v.at[par], sem)
        pltpu.async_copy(qflat_hbm.at[qidx_v.at[par]], qbuf_v.at[par], sem)

    def proc(g, par, half, acc):
        sem = sems[par]
        pltpu.make_async_copy(pflat_hbm.at[pl.ds(0, G * L)], pbuf_v.at[par],
                              sem).wait()
        pltpu.make_async_copy(qflat_hbm.at[pl.ds(0, G * L)], qbuf_v.at[par],
                              sem).wait()
        pb = pbuf_v.at[par]
        qb = qbuf_v.at[par]
        for j in range(G):
            pv = pb[pl.ds(j * L, L)]
            qv = qb[pl.ds(j * L, L)]
            s = jnp.sum(pv * qv)
            acc = jnp.where(lane == half + j, acc + s, acc)
        return acc

    for d in gdescs:
        d.wait()

    for p in range(NPAR - 1):
        fire(p, p)

    def quad(k, carry):
        g0 = 4 * k
        blk0 = pl.ds(2 * k * L, L)
        blk1 = pl.ds((2 * k + 1) * L, L)
        acc0 = bu_v[blk0] + bi_v[blk0]
        acc1 = bu_v[blk1] + bi_v[blk1]
        for jj in range(4):
            g = g0 + jj

            @pl.when(g + NPAR - 1 < NG)
            def _(g=g, jj=jj):
                fire(g + NPAR - 1, (jj + NPAR - 1) % NPAR)

            if jj < 2:
                acc0 = proc(g, jj, (jj % 2) * G, acc0)
            else:
                acc1 = proc(g, jj, (jj % 2) * G, acc1)
        out_v[blk0] = acc0
        out_v[blk1] = acc1
        return carry

    lax.fori_loop(0, NG // 4, quad, 0)
    pltpu.sync_copy(out_v, out_hbm.at[pl.ds(base, BPW)])


def kernel(user_id, item_id, P, Q, Bu, Bi):
    u = user_id.reshape(-1)
    i = item_id.reshape(-1)
    bu = Bu.reshape(-1)
    bi = Bi.reshape(-1)
    p3 = P.T.reshape(2, 8, 1000000)
    q3 = Q.T.reshape(2, 8, 1000000)
    mesh = plsc.VectorSubcoreMesh(core_axis_name="c", subcore_axis_name="s",
                                  num_cores=NC, num_subcores=NS)
    cp = pltpu.CompilerParams(needs_layout_passes=False)

    pout, qout = pl.kernel(
        _linearize_body,
        out_type=(jax.ShapeDtypeStruct((2, 8, RTP), jnp.float32),
                  jax.ShapeDtypeStruct((2, 8, RTP), jnp.float32)),
        mesh=mesh,
        compiler_params=cp,
        scratch_types=[
            pltpu.VMEM((2, 8, CT * 128), jnp.float32),
            pltpu.SemaphoreType.DMA,
            pltpu.SemaphoreType.DMA,
            pltpu.SemaphoreType.DMA,
            pltpu.SemaphoreType.DMA,
        ],
    )(p3, q3)

    def flat(t):
        return (t.reshape(2, 8, RT, 128)
                .transpose(0, 2, 1, 3)
                .reshape(NFLAT))

    out = pl.kernel(
        _gather_body,
        out_type=jax.ShapeDtypeStruct((B,), jnp.float32),
        mesh=mesh,
        compiler_params=cp,
        scratch_types=[
            pltpu.VMEM((BPW + L,), jnp.int32),
            pltpu.VMEM((BPW + L,), jnp.int32),
            pltpu.VMEM((NPAR, G * L), jnp.int32),
            pltpu.VMEM((NPAR, G * L), jnp.int32),
            pltpu.VMEM((NPAR, G * L), jnp.float32),
            pltpu.VMEM((NPAR, G * L), jnp.float32),
            pltpu.VMEM((BPW,), jnp.float32),
            pltpu.VMEM((BPW,), jnp.float32),
            pltpu.VMEM((BPW,), jnp.float32),
            pltpu.SemaphoreType.DMA,
            pltpu.SemaphoreType.DMA,
            pltpu.SemaphoreType.DMA,
            pltpu.SemaphoreType.DMA,
            pltpu.SemaphoreType.DMA,
        ],
    )(u, i, flat(pout), flat(qout), bu, bi)
    return out.reshape(B, 1)


# speedup vs baseline: 28.2042x; 1.0005x over previous
import jax
import jax.numpy as jnp
from jax import lax
from jax.experimental import pallas as pl
from jax.experimental.pallas import tpu as pltpu, tpu_sc as plsc

NC = 2
NS = 16
L = 16
NW = NC * NS
B = 16384
F = 16
BPW = B // NW
CHUNK = 128
NCHUNK = BPW // CHUNK
G = 8
NG = BPW // G
NPAR = 4
RT = 7813
RTP = RT * 128
HALF = RT * 1024
NFLAT = 2 * HALF

NT_W = 245
NCH = 6
CT = 41
NRING = 3


def _linearize_body(p3_hbm, q3_hbm, pout_hbm, qout_hbm, buf_v,
                    isem0, isem1, isem2, osem0, osem1, osem2):
    wid = lax.axis_index("s") * NC + lax.axis_index("c")
    t0 = wid * NT_W
    isems = (isem0, isem1, isem2)
    osems = (osem0, osem1, osem2)
    chunks = []
    for (src3, dst3) in ((p3_hbm, pout_hbm), (q3_hbm, qout_hbm)):
        for g in range(2):
            for c in range(NCH):
                chunks.append((src3, dst3, g, c))

    def s128_of(c):
        start = jnp.minimum(t0 + c * CT, RT - CT)
        return pl.multiple_of(start * 128, 128)

    def start_in(idx, par):
        src3, _, g, c = chunks[idx]
        pltpu.async_copy(src3.at[g, :, pl.ds(s128_of(c), CT * 128)],
                         buf_v.at[par], isems[par])

    def wait_in(par):
        pltpu.make_async_copy(p3_hbm.at[0, :, pl.ds(0, CT * 128)],
                              buf_v.at[par], isems[par]).wait()

    def start_out(idx, par):
        _, dst3, g, c = chunks[idx]
        pltpu.async_copy(buf_v.at[par],
                         dst3.at[g, :, pl.ds(s128_of(c), CT * 128)],
                         osems[par])

    def wait_out(par):
        pltpu.make_async_copy(buf_v.at[par],
                              pout_hbm.at[0, :, pl.ds(0, CT * 128)],
                              osems[par]).wait()

    n = len(chunks)
    start_in(0, 0)
    start_in(1, 1)
    for c in range(n):
        par = c % NRING
        wait_in(par)
        start_out(c, par)
        if c + 2 < n:
            if c >= 1:
                wait_out((c + 2) % NRING)
            start_in(c + 2, (c + 2) % NRING)
    for t in range(NRING):
        wait_out((n - 1 - t) % NRING)


def _gather_body(u_hbm, i_hbm, pflat_hbm, qflat_hbm, bu_hbm, bi_hbm, out_hbm,
                 uidx_v, iidx_v, pidx_v, qidx_v, pbuf_v, qbuf_v,
                 bu_v, bi_v, out_v, sem0, sem1, sem2, sem3, gsem):
    wid = lax.axis_index("s") * NC + lax.axis_index("c")
    base = wid * BPW
    pltpu.sync_copy(u_hbm.at[pl.ds(base, BPW)], uidx_v.at[pl.ds(0, BPW)])
    pltpu.sync_copy(i_hbm.at[pl.ds(base, BPW)], iidx_v.at[pl.ds(0, BPW)])

    gdescs = []
    for j in range(NCHUNK):
        s = pl.ds(j * CHUNK, CHUNK)
        gdescs.append(pltpu.async_copy(bu_hbm.at[uidx_v.at[s]], bu_v.at[s], gsem))
        gdescs.append(pltpu.async_copy(bi_hbm.at[iidx_v.at[s]], bi_v.at[s], gsem))

    lane = lax.iota(jnp.int32, L)
    K = (lane >> 3) * HALF + (lane & 7) * 128
    sems = (sem0, sem1, sem2, sem3)

    def fire(g, par):
        uv = uidx_v[pl.ds(g * G, L)]
        iv = iidx_v[pl.ds(g * G, L)]
        pr = pidx_v.at[par]
        qr = qidx_v.at[par]
        for j in range(G):
            pr[pl.ds(j * L, L)] = K + ((uv[j] >> 7) * 1024 + (uv[j] & 127))
            qr[pl.ds(j * L, L)] = K + ((iv[j] >> 7) * 1024 + (iv[j] & 127))
        sem = sems[par]
        pltpu.async_copy(pflat_hbm.at[pidx_v.at[par]], pbuf_v.at[par], sem)
        pltpu.async_copy(qflat_hbm.at[qidx_v.at[par]], qbuf_v.at[par], sem)

    def proc(g, par, half, acc):
        sem = sems[par]
        pltpu.make_async_copy(pflat_hbm.at[pl.ds(0, G * L)], pbuf_v.at[par],
                              sem).wait()
        pltpu.make_async_copy(qflat_hbm.at[pl.ds(0, G * L)], qbuf_v.at[par],
                              sem).wait()
        pb = pbuf_v.at[par]
        qb = qbuf_v.at[par]
        for j in range(G):
            pv = pb[pl.ds(j * L, L)]
            qv = qb[pl.ds(j * L, L)]
            s = jnp.sum(pv * qv)
            acc = jnp.where(lane == half + j, acc + s, acc)
        return acc

    for d in gdescs:
        d.wait()

    for p in range(NPAR - 1):
        fire(p, p)

    def quad(k, carry):
        g0 = 4 * k
        blk0 = pl.ds(2 * k * L, L)
        blk1 = pl.ds((2 * k + 1) * L, L)
        acc0 = bu_v[blk0] + bi_v[blk0]
        acc1 = bu_v[blk1] + bi_v[blk1]
        for jj in range(4):
            g = g0 + jj

            @pl.when(g + NPAR - 1 < NG)
            def _(g=g, jj=jj):
                fire(g + NPAR - 1, (jj + NPAR - 1) % NPAR)

            if jj < 2:
                acc0 = proc(g, jj, (jj % 2) * G, acc0)
            else:
                acc1 = proc(g, jj, (jj % 2) * G, acc1)
        out_v[blk0] = acc0
        out_v[blk1] = acc1
        return carry

    lax.fori_loop(0, NG // 4, quad, 0)
    pltpu.sync_copy(out_v, out_hbm.at[pl.ds(base, BPW)])


def kernel(user_id, item_id, P, Q, Bu, Bi):
    u = user_id.reshape(-1)
    i = item_id.reshape(-1)
    bu = Bu.reshape(-1)
    bi = Bi.reshape(-1)
    p3 = P.T.reshape(2, 8, 1000000)
    q3 = Q.T.reshape(2, 8, 1000000)
    mesh = plsc.VectorSubcoreMesh(core_axis_name="c", subcore_axis_name="s",
                                  num_cores=NC, num_subcores=NS)
    cp = pltpu.CompilerParams(needs_layout_passes=False)

    pout, qout = pl.kernel(
        _linearize_body,
        out_type=(jax.ShapeDtypeStruct((2, 8, RTP), jnp.float32),
                  jax.ShapeDtypeStruct((2, 8, RTP), jnp.float32)),
        mesh=mesh,
        compiler_params=cp,
        scratch_types=[
            pltpu.VMEM((NRING, 8, CT * 128), jnp.float32),
            pltpu.SemaphoreType.DMA,
            pltpu.SemaphoreType.DMA,
            pltpu.SemaphoreType.DMA,
            pltpu.SemaphoreType.DMA,
            pltpu.SemaphoreType.DMA,
            pltpu.SemaphoreType.DMA,
        ],
    )(p3, q3)

    def flat(t):
        return (t.reshape(2, 8, RT, 128)
                .transpose(0, 2, 1, 3)
                .reshape(NFLAT))

    out = pl.kernel(
        _gather_body,
        out_type=jax.ShapeDtypeStruct((B,), jnp.float32),
        mesh=mesh,
        compiler_params=cp,
        scratch_types=[
            pltpu.VMEM((BPW + L,), jnp.int32),
            pltpu.VMEM((BPW + L,), jnp.int32),
            pltpu.VMEM((NPAR, G * L), jnp.int32),
            pltpu.VMEM((NPAR, G * L), jnp.int32),
            pltpu.VMEM((NPAR, G * L), jnp.float32),
            pltpu.VMEM((NPAR, G * L), jnp.float32),
            pltpu.VMEM((BPW,), jnp.float32),
            pltpu.VMEM((BPW,), jnp.float32),
            pltpu.VMEM((BPW,), jnp.float32),
            pltpu.SemaphoreType.DMA,
            pltpu.SemaphoreType.DMA,
            pltpu.SemaphoreType.DMA,
            pltpu.SemaphoreType.DMA,
            pltpu.SemaphoreType.DMA,
        ],
    )(u, i, flat(pout), flat(qout), bu, bi)
    return out.reshape(B, 1)


# speedup vs baseline: 28.7364x; 1.0189x over previous
import jax
import jax.numpy as jnp
from jax import lax
from jax.experimental import pallas as pl
from jax.experimental.pallas import tpu as pltpu, tpu_sc as plsc

NC = 2
NS = 16
L = 16
NW = NC * NS
B = 16384
F = 16
BPW = B // NW
CHUNK = 128
NCHUNK = BPW // CHUNK
G = 8
NG = BPW // G
NPAR = 4
RT = 7813
RTP = RT * 128
HALF = RT * 1024
NFLAT = 2 * HALF

NT_W = 245
NCH = 6
CT = 41
NRING = 3


def _linearize_body(p3_hbm, q3_hbm, pout_hbm, qout_hbm, sbuf_v,
                    isem0, isem1, isem2, osem0, osem1, osem2):
    wid = lax.axis_index("s") * NC + lax.axis_index("c")
    sid = lax.axis_index("s")
    buf_v = sbuf_v.at[sid]
    t0 = wid * NT_W
    isems = (isem0, isem1, isem2)
    osems = (osem0, osem1, osem2)
    chunks = []
    for (src3, dst3) in ((p3_hbm, pout_hbm), (q3_hbm, qout_hbm)):
        for g in range(2):
            for c in range(NCH):
                chunks.append((src3, dst3, g, c))

    def s128_of(c):
        start = jnp.minimum(t0 + c * CT, RT - CT)
        return pl.multiple_of(start * 128, 128)

    def start_in(idx, par):
        src3, _, g, c = chunks[idx]
        pltpu.async_copy(src3.at[g, :, pl.ds(s128_of(c), CT * 128)],
                         buf_v.at[par], isems[par])

    def wait_in(par):
        pltpu.make_async_copy(p3_hbm.at[0, :, pl.ds(0, CT * 128)],
                              buf_v.at[par], isems[par]).wait()

    def start_out(idx, par):
        _, dst3, g, c = chunks[idx]
        pltpu.async_copy(buf_v.at[par],
                         dst3.at[g, :, pl.ds(s128_of(c), CT * 128)],
                         osems[par])

    def wait_out(par):
        pltpu.make_async_copy(buf_v.at[par],
                              pout_hbm.at[0, :, pl.ds(0, CT * 128)],
                              osems[par]).wait()

    n = len(chunks)
    start_in(0, 0)
    start_in(1, 1)
    for c in range(n):
        par = c % NRING
        wait_in(par)
        start_out(c, par)
        if c + 2 < n:
            if c >= 1:
                wait_out((c + 2) % NRING)
            start_in(c + 2, (c + 2) % NRING)
    for t in range(NRING):
        wait_out((n - 1 - t) % NRING)


def _gather_body(u_hbm, i_hbm, pflat_hbm, qflat_hbm, bu_hbm, bi_hbm, out_hbm,
                 uidx_v, iidx_v, pidx_v, qidx_v, pbuf_v, qbuf_v,
                 bu_v, bi_v, out_v, sem0, sem1, sem2, sem3, gsem):
    wid = lax.axis_index("s") * NC + lax.axis_index("c")
    base = wid * BPW
    pltpu.sync_copy(u_hbm.at[pl.ds(base, BPW)], uidx_v.at[pl.ds(0, BPW)])
    pltpu.sync_copy(i_hbm.at[pl.ds(base, BPW)], iidx_v.at[pl.ds(0, BPW)])

    gdescs = []
    for j in range(NCHUNK):
        s = pl.ds(j * CHUNK, CHUNK)
        gdescs.append(pltpu.async_copy(bu_hbm.at[uidx_v.at[s]], bu_v.at[s], gsem))
        gdescs.append(pltpu.async_copy(bi_hbm.at[iidx_v.at[s]], bi_v.at[s], gsem))

    lane = lax.iota(jnp.int32, L)
    K = (lane >> 3) * HALF + (lane & 7) * 128
    sems = (sem0, sem1, sem2, sem3)

    def fire(g, par):
        uv = uidx_v[pl.ds(g * G, L)]
        iv = iidx_v[pl.ds(g * G, L)]
        pr = pidx_v.at[par]
        qr = qidx_v.at[par]
        for j in range(G):
            pr[pl.ds(j * L, L)] = K + ((uv[j] >> 7) * 1024 + (uv[j] & 127))
            qr[pl.ds(j * L, L)] = K + ((iv[j] >> 7) * 1024 + (iv[j] & 127))
        sem = sems[par]
        pltpu.async_copy(pflat_hbm.at[pidx_v.at[par]], pbuf_v.at[par], sem)
        pltpu.async_copy(qflat_hbm.at[qidx_v.at[par]], qbuf_v.at[par], sem)

    def proc(g, par, half, acc):
        sem = sems[par]
        pltpu.make_async_copy(pflat_hbm.at[pl.ds(0, G * L)], pbuf_v.at[par],
                              sem).wait()
        pltpu.make_async_copy(qflat_hbm.at[pl.ds(0, G * L)], qbuf_v.at[par],
                              sem).wait()
        pb = pbuf_v.at[par]
        qb = qbuf_v.at[par]
        for j in range(G):
            pv = pb[pl.ds(j * L, L)]
            qv = qb[pl.ds(j * L, L)]
            s = jnp.sum(pv * qv)
            acc = jnp.where(lane == half + j, acc + s, acc)
        return acc

    for d in gdescs:
        d.wait()

    for p in range(NPAR - 1):
        fire(p, p)

    def quad(k, carry):
        g0 = 4 * k
        blk0 = pl.ds(2 * k * L, L)
        blk1 = pl.ds((2 * k + 1) * L, L)
        acc0 = bu_v[blk0] + bi_v[blk0]
        acc1 = bu_v[blk1] + bi_v[blk1]
        for jj in range(4):
            g = g0 + jj

            @pl.when(g + NPAR - 1 < NG)
            def _(g=g, jj=jj):
                fire(g + NPAR - 1, (jj + NPAR - 1) % NPAR)

            if jj < 2:
                acc0 = proc(g, jj, (jj % 2) * G, acc0)
            else:
                acc1 = proc(g, jj, (jj % 2) * G, acc1)
        out_v[blk0] = acc0
        out_v[blk1] = acc1
        return carry

    lax.fori_loop(0, NG // 4, quad, 0)
    pltpu.sync_copy(out_v, out_hbm.at[pl.ds(base, BPW)])


def kernel(user_id, item_id, P, Q, Bu, Bi):
    u = user_id.reshape(-1)
    i = item_id.reshape(-1)
    bu = Bu.reshape(-1)
    bi = Bi.reshape(-1)
    p3 = P.T.reshape(2, 8, 1000000)
    q3 = Q.T.reshape(2, 8, 1000000)
    mesh = plsc.VectorSubcoreMesh(core_axis_name="c", subcore_axis_name="s",
                                  num_cores=NC, num_subcores=NS)
    cp = pltpu.CompilerParams(needs_layout_passes=False)

    pout, qout = pl.kernel(
        _linearize_body,
        out_type=(jax.ShapeDtypeStruct((2, 8, RTP), jnp.float32),
                  jax.ShapeDtypeStruct((2, 8, RTP), jnp.float32)),
        mesh=mesh,
        compiler_params=cp,
        scratch_types=[
            pltpu.VMEM_SHARED((NS, NRING, 8, CT * 128), jnp.float32),
            pltpu.SemaphoreType.DMA,
            pltpu.SemaphoreType.DMA,
            pltpu.SemaphoreType.DMA,
            pltpu.SemaphoreType.DMA,
            pltpu.SemaphoreType.DMA,
            pltpu.SemaphoreType.DMA,
        ],
    )(p3, q3)

    def flat(t):
        return (t.reshape(2, 8, RT, 128)
                .transpose(0, 2, 1, 3)
                .reshape(NFLAT))

    out = pl.kernel(
        _gather_body,
        out_type=jax.ShapeDtypeStruct((B,), jnp.float32),
        mesh=mesh,
        compiler_params=cp,
        scratch_types=[
            pltpu.VMEM((BPW + L,), jnp.int32),
            pltpu.VMEM((BPW + L,), jnp.int32),
            pltpu.VMEM((NPAR, G * L), jnp.int32),
            pltpu.VMEM((NPAR, G * L), jnp.int32),
            pltpu.VMEM((NPAR, G * L), jnp.float32),
            pltpu.VMEM((NPAR, G * L), jnp.float32),
            pltpu.VMEM((BPW,), jnp.float32),
            pltpu.VMEM((BPW,), jnp.float32),
            pltpu.VMEM((BPW,), jnp.float32),
            pltpu.SemaphoreType.DMA,
            pltpu.SemaphoreType.DMA,
            pltpu.SemaphoreType.DMA,
            pltpu.SemaphoreType.DMA,
            pltpu.SemaphoreType.DMA,
        ],
    )(u, i, flat(pout), flat(qout), bu, bi)
    return out.reshape(B, 1)
